# Initial kernel scaffold; baseline (speedup 1.0000x reference)
import jax
import jax.numpy as jnp
from jax.experimental import pallas as pl


def kernel(x, edge_index, W1, b1, W2, b2, W3, b3, Wc, bc):
    raise NotImplementedError("write your pallas kernel here")


# speedup vs baseline: 12.0456x; 12.0456x over previous
import jax
import jax.numpy as jnp
from jax import lax
from jax.experimental import pallas as pl
from jax.experimental.pallas import tpu as pltpu
from jax.experimental.pallas import tpu_sc as plsc

_N = 10000
_E = 320000
_H = 128
_CH = 128
_NCH = 2560
_E_PAD = _NCH * _CH
_ACC_ROWS = 10240
_ZPT = _ACC_ROWS // 16
_OPT = 624
_BR = 1000


def _make_agg(F, feature_split, pass_idx=0):
  mesh = plsc.VectorSubcoreMesh(core_axis_name="c", subcore_axis_name="s")
  cpt = (_NCH // 16) if feature_split else (_NCH // 32)

  def body(y_hbm, src_hbm, dst_hbm, out_hbm, src_v, dst_v, buf0, buf1, acc,
           sem0, sem1):
    c = lax.axis_index("c")
    s = lax.axis_index("s")
    if feature_split:
      base = s * cpt
    else:
      base = c * (_NCH // 2) + s * cpt
    pltpu.sync_copy(src_hbm.at[pl.ds(base, cpt)], src_v)
    pltpu.sync_copy(dst_hbm.at[pl.ds(base, cpt)], dst_v)

    if feature_split:
      offv = jnp.zeros((16,), jnp.int32) + (2 * pass_idx + c) * _N

      @pl.loop(0, cpt)
      def _addoff(i):
        for j in range(_CH // 16):
          src_v[i, pl.ds(j * 16, 16)] = src_v[i, pl.ds(j * 16, 16)] + offv

    @pl.loop(0, _CH)
    def _zero_fill(i):
      for j in range(F // 16):
        buf0[i, pl.ds(j * 16, 16)] = jnp.zeros((16,), jnp.float32)

    z0 = s * _ZPT
    for r in range(_ZPT // _CH):
      pltpu.sync_copy(buf0, acc.at[pl.ds(z0 + r * _CH, _CH)])
    plsc.subcore_barrier()

    pltpu.async_copy(y_hbm.at[src_v.at[0]], buf0, sem0)
    pltpu.async_copy(y_hbm.at[src_v.at[1]], buf1, sem1)

    @pl.loop(0, cpt // 2)
    def _chunks(k):
      j = k * 2
      pltpu.make_async_copy(y_hbm.at[src_v.at[0]], buf0, sem0).wait()
      pltpu.sync_copy(buf0, acc.at[dst_v.at[j]], add=True)

      @pl.when(j + 2 < cpt)
      def _():
        pltpu.async_copy(y_hbm.at[src_v.at[j + 2]], buf0, sem0)

      pltpu.make_async_copy(y_hbm.at[src_v.at[0]], buf1, sem1).wait()
      pltpu.sync_copy(buf1, acc.at[dst_v.at[j + 1]], add=True)

      @pl.when(j + 3 < cpt)
      def _():
        pltpu.async_copy(y_hbm.at[src_v.at[j + 3]], buf1, sem1)

    plsc.subcore_barrier()
    o0 = s * _OPT
    pltpu.sync_copy(acc.at[pl.ds(o0, _OPT)], out_hbm.at[c, pl.ds(o0, _OPT)])

    @pl.when(s == 15)
    def _tail():
      t0 = 16 * _OPT
      pltpu.sync_copy(acc.at[pl.ds(t0, _N - 16 * _OPT)],
                      out_hbm.at[c, pl.ds(t0, _N - 16 * _OPT)])

  return pl.kernel(
      body,
      out_type=jax.ShapeDtypeStruct((2, _N, F), jnp.float32),
      mesh=mesh,
      compiler_params=pltpu.CompilerParams(use_tc_tiling_on_sc=False),
      scratch_types=[
          pltpu.VMEM((cpt, _CH), jnp.int32),
          pltpu.VMEM((cpt, _CH), jnp.int32),
          pltpu.VMEM((_CH, F), jnp.float32),
          pltpu.VMEM((_CH, F), jnp.float32),
          pltpu.VMEM_SHARED((_ACC_ROWS, F), jnp.float32),
          pltpu.SemaphoreType.DMA,
          pltpu.SemaphoreType.DMA,
      ],
  )


def _t_first(degp, x, w1):
  def body(dp, xr, wr, y_ref, dis_ref):
    v = dp[...]
    deg = v[0, :, 0:1] + v[1, :, 0:1] + 1.0
    dis = lax.rsqrt(deg)
    xw = jnp.dot(xr[...], wr[...], preferred_element_type=jnp.float32) * dis
    y_ref[...] = jnp.stack(
        [xw[:, 32 * g:32 * g + 32] for g in range(4)], axis=0)
    dis_ref[...] = dis

  return pl.pallas_call(
      body,
      grid=(_N // _BR,),
      in_specs=[
          pl.BlockSpec((2, _BR, 16), lambda i: (0, i, 0)),
          pl.BlockSpec((_BR, _H), lambda i: (i, 0)),
          pl.BlockSpec((_H, _H), lambda i: (0, 0)),
      ],
      out_specs=[
          pl.BlockSpec((4, _BR, 32), lambda i: (0, i, 0)),
          pl.BlockSpec((_BR, 1), lambda i: (i, 0)),
      ],
      out_shape=[
          jax.ShapeDtypeStruct((4, _N, 32), jnp.float32),
          jax.ShapeDtypeStruct((_N, 1), jnp.float32),
      ],
  )(degp, x, w1)


def _t_mid(pa, pb, y, dis, b, w, grouped_out):
  def body(pr_a, pr_b, yr, dr, br, wr, o_ref):
    va, vb, vy = pr_a[...], pr_b[...], yr[...]
    agg = jnp.concatenate(
        [va[0] + vy[0], va[1] + vy[1], vb[0] + vy[2], vb[1] + vy[3]], axis=1)
    dis_v = dr[...]
    h = jnp.maximum(agg * dis_v + br[...], 0.0)
    hw = jnp.dot(h, wr[...], preferred_element_type=jnp.float32) * dis_v
    if grouped_out:
      o_ref[...] = jnp.stack(
          [hw[:, 32 * g:32 * g + 32] for g in range(4)], axis=0)
    else:
      o_ref[...] = hw

  fo = w.shape[1]
  if grouped_out:
    out_spec = pl.BlockSpec((4, _BR, 32), lambda i: (0, i, 0))
    out_shape = jax.ShapeDtypeStruct((4, _N, 32), jnp.float32)
  else:
    out_spec = pl.BlockSpec((_BR, fo), lambda i: (i, 0))
    out_shape = jax.ShapeDtypeStruct((_N, fo), jnp.float32)
  return pl.pallas_call(
      body,
      grid=(_N // _BR,),
      in_specs=[
          pl.BlockSpec((2, _BR, 32), lambda i: (0, i, 0)),
          pl.BlockSpec((2, _BR, 32), lambda i: (0, i, 0)),
          pl.BlockSpec((4, _BR, 32), lambda i: (0, i, 0)),
          pl.BlockSpec((_BR, 1), lambda i: (i, 0)),
          pl.BlockSpec((1, _H), lambda i: (0, 0)),
          pl.BlockSpec((_H, fo), lambda i: (0, 0)),
      ],
      out_specs=out_spec,
      out_shape=out_shape,
  )(pa, pb, y, dis, b, w)


def _t_last(p, y, dis, b3p, wcp, bcr):
  def body(pr, yr, dr, br, wr, bcref, o_ref):
    v = pr[...]
    h = jnp.maximum((v[0] + v[1] + yr[...]) * dr[...] + br[...], 0.0)
    o_ref[...] = jnp.dot(h, wr[...],
                         preferred_element_type=jnp.float32) + bcref[...]

  return pl.pallas_call(
      body,
      grid=(_N // _BR,),
      in_specs=[
          pl.BlockSpec((2, _BR, 16), lambda i: (0, i, 0)),
          pl.BlockSpec((_BR, 16), lambda i: (i, 0)),
          pl.BlockSpec((_BR, 1), lambda i: (i, 0)),
          pl.BlockSpec((1, 16), lambda i: (0, 0)),
          pl.BlockSpec((16, 16), lambda i: (0, 0)),
          pl.BlockSpec((1, 16), lambda i: (0, 0)),
      ],
      out_specs=pl.BlockSpec((_BR, 16), lambda i: (i, 0)),
      out_shape=jax.ShapeDtypeStruct((_N, 16), jnp.float32),
  )(p, y, dis, b3p, wcp, bcr)


def kernel(x, edge_index, W1, b1, W2, b2, W3, b3, Wc, bc):
  src = edge_index[0]
  dst = edge_index[1]
  pad = _E_PAD - _E
  src_p = jnp.concatenate(
      [src, jnp.zeros((pad,), jnp.int32)]).reshape(_NCH, _CH)
  dst_p = jnp.concatenate(
      [dst, jnp.full((pad,), _N, jnp.int32)]).reshape(_NCH, _CH)

  agg16 = _make_agg(16, feature_split=False)
  agg32_0 = _make_agg(32, feature_split=True, pass_idx=0)
  agg32_1 = _make_agg(32, feature_split=True, pass_idx=1)

  def agg_layer(y_grouped):
    yt = y_grouped.reshape(4 * _N, 32)
    return agg32_0(yt, src_p, dst_p), agg32_1(yt, src_p, dst_p)

  degp = agg16(jnp.ones((_N, 16), jnp.float32), src_p, dst_p)
  y1, dis = _t_first(degp, x, W1)
  p1a, p1b = agg_layer(y1)
  y2 = _t_mid(p1a, p1b, y1, dis, b1.reshape(1, _H), W2, grouped_out=True)
  p2a, p2b = agg_layer(y2)
  w3p = jnp.pad(W3, ((0, 0), (0, 8)))
  y3 = _t_mid(p2a, p2b, y2, dis, b2.reshape(1, _H), w3p, grouped_out=False)
  p3 = agg16(y3, src_p, dst_p)
  out = _t_last(p3, y3, dis,
                jnp.pad(b3, (0, 8)).reshape(1, 16),
                jnp.pad(Wc, ((0, 8), (0, 0))),
                bc.reshape(1, 16))
  return out


# speedup vs baseline: 12.4950x; 1.0373x over previous
import jax
import jax.numpy as jnp
from jax import lax
from jax.experimental import pallas as pl
from jax.experimental.pallas import tpu as pltpu
from jax.experimental.pallas import tpu_sc as plsc

_N = 10000
_E = 320000
_H = 128
_CH = 128
_NCH = 2560
_E_PAD = _NCH * _CH
_ACC_ROWS = 10240
_ZPT = _ACC_ROWS // 16
_OPT = 624
_BR = 1000


def _make_agg(F, feature_split, pass_idx=0):
  mesh = plsc.VectorSubcoreMesh(core_axis_name="c", subcore_axis_name="s")
  cpt = (_NCH // 16) if feature_split else (_NCH // 32)

  def body(y_hbm, src_hbm, dst_hbm, out_hbm, src_v, dst_v, b0, b1, b2, b3,
           acc, g0, g1, g2, g3, s0, s1, s2, s3):
    bufs = (b0, b1, b2, b3)
    gsem = (g0, g1, g2, g3)
    ssem = (s0, s1, s2, s3)
    c = lax.axis_index("c")
    s = lax.axis_index("s")
    if feature_split:
      base = s * cpt
    else:
      base = c * (_NCH // 2) + s * cpt
    pltpu.sync_copy(src_hbm.at[pl.ds(base, cpt)], src_v)
    pltpu.sync_copy(dst_hbm.at[pl.ds(base, cpt)], dst_v)

    if feature_split:
      offv = jnp.zeros((16,), jnp.int32) + (2 * pass_idx + c) * _N

      @pl.loop(0, cpt)
      def _addoff(i):
        for j in range(_CH // 16):
          src_v[i, pl.ds(j * 16, 16)] = src_v[i, pl.ds(j * 16, 16)] + offv

    @pl.loop(0, _CH)
    def _zero_fill(i):
      for j in range(F // 16):
        b0[i, pl.ds(j * 16, 16)] = jnp.zeros((16,), jnp.float32)

    z0 = s * _ZPT
    for r in range(_ZPT // _CH):
      pltpu.sync_copy(b0, acc.at[pl.ds(z0 + r * _CH, _CH)])
    plsc.subcore_barrier()

    for b in range(4):
      pltpu.async_copy(y_hbm.at[src_v.at[b]], bufs[b], gsem[b])

    @pl.loop(0, cpt // 4)
    def _chunks(k):
      j0 = k * 4
      for b in range(4):
        pltpu.make_async_copy(y_hbm.at[src_v.at[0]], bufs[b], gsem[b]).wait()
        pltpu.async_copy(bufs[b], acc.at[dst_v.at[j0 + b]], ssem[b],
                         add=True)
      for b in range(4):

        @pl.when(j0 + b + 4 < cpt)
        def _():
          pltpu.make_async_copy(bufs[b], acc.at[dst_v.at[0]], ssem[b]).wait()
          pltpu.async_copy(y_hbm.at[src_v.at[j0 + b + 4]], bufs[b], gsem[b])

    for b in range(4):
      pltpu.make_async_copy(bufs[b], acc.at[dst_v.at[0]], ssem[b]).wait()
    plsc.subcore_barrier()
    o0 = s * _OPT
    pltpu.sync_copy(acc.at[pl.ds(o0, _OPT)], out_hbm.at[c, pl.ds(o0, _OPT)])

    @pl.when(s == 15)
    def _tail():
      t0 = 16 * _OPT
      pltpu.sync_copy(acc.at[pl.ds(t0, _N - 16 * _OPT)],
                      out_hbm.at[c, pl.ds(t0, _N - 16 * _OPT)])

  return pl.kernel(
      body,
      out_type=jax.ShapeDtypeStruct((2, _N, F), jnp.float32),
      mesh=mesh,
      compiler_params=pltpu.CompilerParams(use_tc_tiling_on_sc=False),
      scratch_types=[
          pltpu.VMEM((cpt, _CH), jnp.int32),
          pltpu.VMEM((cpt, _CH), jnp.int32),
          pltpu.VMEM((_CH, F), jnp.float32),
          pltpu.VMEM((_CH, F), jnp.float32),
          pltpu.VMEM((_CH, F), jnp.float32),
          pltpu.VMEM((_CH, F), jnp.float32),
          pltpu.VMEM_SHARED((_ACC_ROWS, F), jnp.float32),
      ] + [pltpu.SemaphoreType.DMA] * 8,
  )


def _t_first(degp, x, w1):
  def body(dp, xr, wr, y_ref, dis_ref):
    v = dp[...]
    deg = v[0, :, 0:1] + v[1, :, 0:1] + 1.0
    dis = lax.rsqrt(deg)
    xw = jnp.dot(xr[...], wr[...], preferred_element_type=jnp.float32) * dis
    y_ref[...] = jnp.stack(
        [xw[:, 32 * g:32 * g + 32] for g in range(4)], axis=0)
    dis_ref[...] = dis

  return pl.pallas_call(
      body,
      grid=(_N // _BR,),
      in_specs=[
          pl.BlockSpec((2, _BR, 16), lambda i: (0, i, 0)),
          pl.BlockSpec((_BR, _H), lambda i: (i, 0)),
          pl.BlockSpec((_H, _H), lambda i: (0, 0)),
      ],
      out_specs=[
          pl.BlockSpec((4, _BR, 32), lambda i: (0, i, 0)),
          pl.BlockSpec((_BR, 1), lambda i: (i, 0)),
      ],
      out_shape=[
          jax.ShapeDtypeStruct((4, _N, 32), jnp.float32),
          jax.ShapeDtypeStruct((_N, 1), jnp.float32),
      ],
  )(degp, x, w1)


def _t_mid(pa, pb, y, dis, b, w, grouped_out):
  def body(pr_a, pr_b, yr, dr, br, wr, o_ref):
    va, vb, vy = pr_a[...], pr_b[...], yr[...]
    agg = jnp.concatenate(
        [va[0] + vy[0], va[1] + vy[1], vb[0] + vy[2], vb[1] + vy[3]], axis=1)
    dis_v = dr[...]
    h = jnp.maximum(agg * dis_v + br[...], 0.0)
    hw = jnp.dot(h, wr[...], preferred_element_type=jnp.float32) * dis_v
    if grouped_out:
      o_ref[...] = jnp.stack(
          [hw[:, 32 * g:32 * g + 32] for g in range(4)], axis=0)
    else:
      o_ref[...] = hw

  fo = w.shape[1]
  if grouped_out:
    out_spec = pl.BlockSpec((4, _BR, 32), lambda i: (0, i, 0))
    out_shape = jax.ShapeDtypeStruct((4, _N, 32), jnp.float32)
  else:
    out_spec = pl.BlockSpec((_BR, fo), lambda i: (i, 0))
    out_shape = jax.ShapeDtypeStruct((_N, fo), jnp.float32)
  return pl.pallas_call(
      body,
      grid=(_N // _BR,),
      in_specs=[
          pl.BlockSpec((2, _BR, 32), lambda i: (0, i, 0)),
          pl.BlockSpec((2, _BR, 32), lambda i: (0, i, 0)),
          pl.BlockSpec((4, _BR, 32), lambda i: (0, i, 0)),
          pl.BlockSpec((_BR, 1), lambda i: (i, 0)),
          pl.BlockSpec((1, _H), lambda i: (0, 0)),
          pl.BlockSpec((_H, fo), lambda i: (0, 0)),
      ],
      out_specs=out_spec,
      out_shape=out_shape,
  )(pa, pb, y, dis, b, w)


def _t_last(p, y, dis, b3p, wcp, bcr):
  def body(pr, yr, dr, br, wr, bcref, o_ref):
    v = pr[...]
    h = jnp.maximum((v[0] + v[1] + yr[...]) * dr[...] + br[...], 0.0)
    o_ref[...] = jnp.dot(h, wr[...],
                         preferred_element_type=jnp.float32) + bcref[...]

  return pl.pallas_call(
      body,
      grid=(_N // _BR,),
      in_specs=[
          pl.BlockSpec((2, _BR, 16), lambda i: (0, i, 0)),
          pl.BlockSpec((_BR, 16), lambda i: (i, 0)),
          pl.BlockSpec((_BR, 1), lambda i: (i, 0)),
          pl.BlockSpec((1, 16), lambda i: (0, 0)),
          pl.BlockSpec((16, 16), lambda i: (0, 0)),
          pl.BlockSpec((1, 16), lambda i: (0, 0)),
      ],
      out_specs=pl.BlockSpec((_BR, 16), lambda i: (i, 0)),
      out_shape=jax.ShapeDtypeStruct((_N, 16), jnp.float32),
  )(p, y, dis, b3p, wcp, bcr)


def kernel(x, edge_index, W1, b1, W2, b2, W3, b3, Wc, bc):
  src = edge_index[0]
  dst = edge_index[1]
  pad = _E_PAD - _E
  src_p = jnp.concatenate(
      [src, jnp.zeros((pad,), jnp.int32)]).reshape(_NCH, _CH)
  dst_p = jnp.concatenate(
      [dst, jnp.full((pad,), _N, jnp.int32)]).reshape(_NCH, _CH)

  agg16 = _make_agg(16, feature_split=False)
  agg32_0 = _make_agg(32, feature_split=True, pass_idx=0)
  agg32_1 = _make_agg(32, feature_split=True, pass_idx=1)

  def agg_layer(y_grouped):
    yt = y_grouped.reshape(4 * _N, 32)
    return agg32_0(yt, src_p, dst_p), agg32_1(yt, src_p, dst_p)

  degp = agg16(jnp.ones((_N, 16), jnp.float32), src_p, dst_p)
  y1, dis = _t_first(degp, x, W1)
  p1a, p1b = agg_layer(y1)
  y2 = _t_mid(p1a, p1b, y1, dis, b1.reshape(1, _H), W2, grouped_out=True)
  p2a, p2b = agg_layer(y2)
  w3p = jnp.pad(W3, ((0, 0), (0, 8)))
  y3 = _t_mid(p2a, p2b, y2, dis, b2.reshape(1, _H), w3p, grouped_out=False)
  p3 = agg16(y3, src_p, dst_p)
  out = _t_last(p3, y3, dis,
                jnp.pad(b3, (0, 8)).reshape(1, 16),
                jnp.pad(Wc, ((0, 8), (0, 0))),
                bc.reshape(1, 16))
  return out
